# baseline (speedup 1.0000x reference)
import jax
import jax.numpy as jnp
from jax import lax
from jax.experimental import pallas as pl
from jax.experimental.pallas import tpu as pltpu
from jax.experimental.pallas import tpu_sc as plsc

N_NODES = 10000
N_EDGES = 320000
D = 128
DE = 4
G = 64
EAP = 16

NC = 2
NS = 16
NW = NC * NS
K = 80
NCHUNK = 128
EPW = NCHUNK * K
EPAD = NW * EPW
NDUM = 240
NP = 10240
ZR = NP // NS
NBUF = 2
GW = 16
NWND = NCHUNK // GW

NB = 2000
NBLK = N_NODES // NB

_f32 = jnp.float32


def _sc_mesh():
  return plsc.VectorSubcoreMesh(
      core_axis_name="c", subcore_axis_name="s", num_cores=NC,
      num_subcores=NS)


def _make_sc_pass(with_ea: bool, with_att: bool):
  out_type = [jax.ShapeDtypeStruct((NC * NP, D), _f32)]
  if with_ea:
    out_type.append(jax.ShapeDtypeStruct((NC * NP, EAP), _f32))

  scratch = [
      pltpu.VMEM_SHARED((NP, D), _f32),
      pltpu.VMEM((2 * GW, K), jnp.int32),
  ]
  scratch += [pltpu.VMEM((K, D), _f32) for _ in range(NBUF)]
  scratch += [pltpu.SemaphoreType.DMA for _ in range(NBUF)]
  if with_ea:
    scratch += [pltpu.VMEM_SHARED((NP, EAP), _f32)]
    scratch += [pltpu.VMEM((K, EAP), _f32) for _ in range(NBUF)]
    scratch += [pltpu.SemaphoreType.DMA for _ in range(NBUF)]
  if with_att:
    scratch += [pltpu.VMEM((K, EAP), _f32) for _ in range(NBUF)]
    scratch += [pltpu.SemaphoreType.DMA for _ in range(NBUF)]

  def body(*refs):
    it = iter(refs)
    y = next(it)
    edges_r = next(it)
    zeros = next(it)
    ea = next(it) if with_ea else None
    z16 = next(it) if with_ea else None
    att = next(it) if with_att else None
    s_out = next(it)
    ea_out = next(it) if with_ea else None
    acc = next(it)
    ib = next(it)
    rows = [next(it) for _ in range(NBUF)]
    sem_r = [next(it) for _ in range(NBUF)]
    if with_ea:
      acc_ea = next(it)
      eab = [next(it) for _ in range(NBUF)]
      sem_e = [next(it) for _ in range(NBUF)]
    if with_att:
      attb = [next(it) for _ in range(NBUF)]
      sem_a = [next(it) for _ in range(NBUF)]

    c = lax.axis_index("c")
    s = lax.axis_index("s")
    wid = s * NC + c
    r0 = s * ZR

    def start_g(m, r):
      b = r % NBUF
      idx = ib.at[2 * r]
      ds = [pltpu.async_copy(y.at[idx], rows[b], sem_r[b])]
      if with_att:
        ds.append(pltpu.async_copy(att.at[idx], attb[b], sem_a[b]))
      if with_ea:
        ds.append(pltpu.async_copy(
            ea.at[pl.ds(wid * EPW + m * (GW * K) + r * K, K)],
            eab[b], sem_e[b]))
      return ds

    def finish(r, ds):
      b = r % NBUF
      for d in ds:
        d.wait()
      if with_att:
        def mul8(t, carry):
          for jj in range(8):
            j = t * 8 + jj
            eab[b][j, :] = eab[b][j, :] * attb[b][j, :]
          return carry

        lax.fori_loop(0, K // 8, mul8, 0)
      didx = ib.at[2 * r + 1]
      pltpu.sync_copy(rows[b], acc.at[didx], add=True)
      if with_ea:
        pltpu.sync_copy(eab[b], acc_ea.at[didx], add=True)

    pltpu.sync_copy(zeros.at[pl.ds(r0, ZR)], acc.at[pl.ds(r0, ZR)])
    if with_ea:
      pltpu.sync_copy(z16.at[pl.ds(r0, ZR)], acc_ea.at[pl.ds(r0, ZR)])

    plsc.subcore_barrier()

    def window(m, carry):
      pltpu.sync_copy(edges_r.at[wid * NWND + m], ib)
      for r in range(GW):
        finish(r, start_g(m, r))
      return carry

    lax.fori_loop(0, NWND, window, 0)

    plsc.subcore_barrier()
    o0 = c * NP + r0
    pltpu.sync_copy(acc.at[pl.ds(r0, ZR)], s_out.at[pl.ds(o0, ZR)])
    if with_ea:
      pltpu.sync_copy(acc_ea.at[pl.ds(r0, ZR)], ea_out.at[pl.ds(o0, ZR)])

  return pl.kernel(
      body, out_type=out_type, mesh=_sc_mesh(),
      scratch_types=scratch,
      compiler_params=pltpu.CompilerParams(use_tc_tiling_on_sc=False))


_sc_pass_plain = _make_sc_pass(with_ea=False, with_att=False)
_sc_pass_ea = _make_sc_pass(with_ea=True, with_att=False)
_sc_pass_ea_att = _make_sc_pass(with_ea=True, with_att=True)


def _dot(a, b):
  return jnp.dot(a, b, preferred_element_type=_f32)


def _part_specs():
  return [
      pl.BlockSpec((1, NB, D), lambda i: (0, i, 0)),
      pl.BlockSpec((1, NB, D), lambda i: (1, i, 0)),
      pl.BlockSpec((1, NB, EAP), lambda i: (0, i, 0)),
      pl.BlockSpec((1, NB, EAP), lambda i: (1, i, 0)),
  ]


def _w_spec(shape):
  return pl.BlockSpec(shape, lambda i: tuple(0 for _ in shape))


def _layer1_body(sa, sb, pa, pb, x, we, w, ws, b, o_h1):
  agg = sa[0] + sb[0] + _dot((pa[0] + pb[0])[:, :DE], we[...])
  o_h1[...] = jax.nn.relu(_dot(agg, w[...]) + _dot(x[...], ws[...]) + b[...])


def _layer2_body(sa, sb, pa, pb, h1, x, we, w, ws, b, we1, be1, we2, be2,
                 o_att, o_y3):
  agg = sa[0] + sb[0] + _dot((pa[0] + pb[0])[:, :DE], we[...])
  emb = jax.nn.relu(_dot(agg, w[...]) + _dot(h1[...], ws[...]) + b[...])
  logit = _dot(jax.nn.relu(_dot(emb, we1[...]) + be1[...]), we2[...]) + be2[...]
  att = jax.nn.sigmoid(logit)
  o_att[...] = jnp.broadcast_to(att, (NB, EAP))
  o_y3[...] = x[...] * att


def _layer3_body(sa, sb, pa, pb, attp, x, we, w, ws, b, o_g1, o_y4):
  att = attp[:, 0:1]
  agg = (sa[0] + sb[0] + _dot((pa[0] + pb[0])[:, :DE], we[...])) * att
  g1 = jax.nn.relu(_dot(agg, w[...]) + _dot(x[...], ws[...]) + b[...])
  o_g1[...] = g1
  o_y4[...] = g1 * att


def _layer4_body(sa, sb, pa, pb, attp, g1, we, w, ws, b, o_g2):
  att = attp[:, 0:1]
  agg = (sa[0] + sb[0] + _dot((pa[0] + pb[0])[:, :DE], we[...])) * att
  o_g2[...] = jax.nn.relu(_dot(agg, w[...]) + _dot(g1[...], ws[...]) + b[...])


def _pool_body(g2, bt, wout, bout, wm, bm, o, acc, cnt):
  i = pl.program_id(0)

  @pl.when(i == 0)
  def _():
    acc[...] = jnp.zeros_like(acc)
    cnt[...] = jnp.zeros_like(cnt)

  gids = lax.broadcasted_iota(jnp.int32, (G, NB), 0)
  mask = (bt[0, 0, :][None, :] == gids).astype(_f32)
  acc[...] += _dot(mask, g2[...])
  cnt[...] += jnp.broadcast_to(jnp.sum(mask, axis=1, keepdims=True), (G, D))

  @pl.when(i == NBLK - 1)
  def _():
    pooled = acc[...] / jnp.maximum(cnt[...], 1.0)
    logits = _dot(pooled, wout[...]) + bout[...]
    o[...] = _dot(logits, wm[...]) + bm[...]


def _tc_layer1(sp, pp, x, we, w, ws, b):
  specs = _part_specs() + [
      pl.BlockSpec((NB, D), lambda i: (i, 0)),
      _w_spec((DE, D)), _w_spec((D, D)), _w_spec((D, D)), _w_spec((1, D)),
  ]
  return pl.pallas_call(
      _layer1_body, grid=(NBLK,), in_specs=specs,
      out_specs=pl.BlockSpec((NB, D), lambda i: (i, 0)),
      out_shape=jax.ShapeDtypeStruct((N_NODES, D), _f32),
  )(sp, sp, pp, pp, x, we, w, ws, b)


def _tc_layer2(sp, pp, h1, x, we, w, ws, b, we1, be1, we2, be2):
  specs = _part_specs() + [
      pl.BlockSpec((NB, D), lambda i: (i, 0)),
      pl.BlockSpec((NB, D), lambda i: (i, 0)),
      _w_spec((DE, D)), _w_spec((D, D)), _w_spec((D, D)), _w_spec((1, D)),
      _w_spec((D, 64)), _w_spec((1, 64)), _w_spec((64, 1)), _w_spec((1, 1)),
  ]
  return pl.pallas_call(
      _layer2_body, grid=(NBLK,), in_specs=specs,
      out_specs=[pl.BlockSpec((NB, EAP), lambda i: (i, 0)),
                 pl.BlockSpec((NB, D), lambda i: (i, 0))],
      out_shape=[jax.ShapeDtypeStruct((N_NODES, EAP), _f32),
                 jax.ShapeDtypeStruct((N_NODES, D), _f32)],
  )(sp, sp, pp, pp, h1, x, we, w, ws, b, we1, be1, we2, be2)


def _tc_layer3(sp, pp, attp, x, we, w, ws, b):
  specs = _part_specs() + [
      pl.BlockSpec((NB, EAP), lambda i: (i, 0)),
      pl.BlockSpec((NB, D), lambda i: (i, 0)),
      _w_spec((DE, D)), _w_spec((D, D)), _w_spec((D, D)), _w_spec((1, D)),
  ]
  return pl.pallas_call(
      _layer3_body, grid=(NBLK,), in_specs=specs,
      out_specs=[pl.BlockSpec((NB, D), lambda i: (i, 0)),
                 pl.BlockSpec((NB, D), lambda i: (i, 0))],
      out_shape=[jax.ShapeDtypeStruct((N_NODES, D), _f32),
                 jax.ShapeDtypeStruct((N_NODES, D), _f32)],
  )(sp, sp, pp, pp, attp, x, we, w, ws, b)


def _tc_layer4(sp, pp, attp, g1, we, w, ws, b):
  specs = _part_specs() + [
      pl.BlockSpec((NB, EAP), lambda i: (i, 0)),
      pl.BlockSpec((NB, D), lambda i: (i, 0)),
      _w_spec((DE, D)), _w_spec((D, D)), _w_spec((D, D)), _w_spec((1, D)),
  ]
  return pl.pallas_call(
      _layer4_body, grid=(NBLK,), in_specs=specs,
      out_specs=pl.BlockSpec((NB, D), lambda i: (i, 0)),
      out_shape=jax.ShapeDtypeStruct((N_NODES, D), _f32),
  )(sp, sp, pp, pp, attp, g1, we, w, ws, b)


def _tc_pool(g2, batch_r, wout, bout, wm, bm):
  specs = [
      pl.BlockSpec((NB, D), lambda i: (i, 0)),
      pl.BlockSpec((1, 1, NB), lambda i: (i, 0, 0)),
      _w_spec((D, 1)), _w_spec((1, 1)), _w_spec((1, 2)), _w_spec((1, 2)),
  ]
  return pl.pallas_call(
      _pool_body, grid=(NBLK,), in_specs=specs,
      out_specs=pl.BlockSpec((G, 2), lambda i: (0, 0)),
      out_shape=jax.ShapeDtypeStruct((G, 2), _f32),
      scratch_shapes=[pltpu.VMEM((G, D), _f32), pltpu.VMEM((G, D), _f32)],
  )(g2, batch_r, wout, bout, wm, bm)


@jax.jit
def kernel(x, edge_index, batch, edge_attr, We1, W1, Ws1, b1, We2, W2, Ws2,
           b2, Wext1, bext1, Wext2, bext2, Wout, bout, Wm, bm):
  src = edge_index[0]
  dst = edge_index[1]
  epw_real = N_EDGES // NW
  dum_src = jnp.zeros((NW, NDUM), jnp.int32)
  dum_dst = jnp.broadcast_to(
      N_NODES + jnp.arange(NDUM, dtype=jnp.int32), (NW, NDUM))
  src_r = jnp.concatenate(
      [src.reshape(NW, epw_real), dum_src], axis=1).reshape(NW * NCHUNK, K)
  dst_r = jnp.concatenate(
      [dst.reshape(NW, epw_real), dum_dst], axis=1).reshape(NW * NCHUNK, K)
  edges_r = jnp.stack([src_r, dst_r], axis=1).reshape(
      NW * NWND, 2 * GW, K)
  ea_pad = jnp.zeros((NW, EPW, EAP), _f32).at[:, :epw_real, :DE].set(
      edge_attr.reshape(NW, epw_real, DE)).reshape(EPAD, EAP)
  zeros = jnp.zeros((NP, D), _f32)
  z16 = jnp.zeros((NP, EAP), _f32)
  b1r = b1.reshape(1, D)
  b2r = b2.reshape(1, D)
  be1 = bext1.reshape(1, 64)
  be2 = bext2.reshape(1, 1)
  boutr = bout.reshape(1, 1)
  bmr = bm.reshape(1, 2)
  batch_r = batch.reshape(NBLK, 1, NB)

  s1, p4a = _sc_pass_ea(x, edges_r, zeros, ea_pad, z16)
  s1 = s1.reshape(NC, NP, D)
  p4a = p4a.reshape(NC, NP, EAP)
  h1 = _tc_layer1(s1, p4a, x, We1, W1, Ws1, b1r)

  (s2,) = _sc_pass_plain(h1, edges_r, zeros)
  s2 = s2.reshape(NC, NP, D)
  attp, y3 = _tc_layer2(s2, p4a, h1, x, We2, W2, Ws2, b2r,
                        Wext1, be1, Wext2, be2)

  s3, p4b = _sc_pass_ea_att(y3, edges_r, zeros, ea_pad, z16, attp)
  s3 = s3.reshape(NC, NP, D)
  p4b = p4b.reshape(NC, NP, EAP)
  g1, y4 = _tc_layer3(s3, p4b, attp, x, We1, W1, Ws1, b1r)

  (s4,) = _sc_pass_plain(y4, edges_r, zeros)
  s4 = s4.reshape(NC, NP, D)
  g2 = _tc_layer4(s4, p4b, attp, g1, We2, W2, Ws2, b2r)

  return _tc_pool(g2, batch_r, Wout, boutr, Wm, bmr)


# speedup vs baseline: 1.1680x; 1.1680x over previous
import jax
import jax.numpy as jnp
from jax import lax
from jax.experimental import pallas as pl
from jax.experimental.pallas import tpu as pltpu
from jax.experimental.pallas import tpu_sc as plsc

N_NODES = 10000
N_EDGES = 320000
D = 128
DE = 4
G = 64
EAP = 16

NC = 2
NS = 16
NW = NC * NS
K = 80
NCHUNK = 128
EPW = NCHUNK * K
EPAD = NW * EPW
NDUM = 240
NP = 10240
ZR = NP // NS
NBUF = 2
NIB = 3
GW = 16
NWND = NCHUNK // GW

NB = 2000
NBLK = N_NODES // NB

_f32 = jnp.float32


def _sc_mesh():
  return plsc.VectorSubcoreMesh(
      core_axis_name="c", subcore_axis_name="s", num_cores=NC,
      num_subcores=NS)


def _make_sc_pass(with_ea: bool, with_att: bool):
  out_type = [jax.ShapeDtypeStruct((NC * NP, D), _f32)]
  if with_ea:
    out_type.append(jax.ShapeDtypeStruct((NC * NP, EAP), _f32))

  scratch = [
      pltpu.VMEM_SHARED((NP, D), _f32),
  ]
  scratch += [pltpu.VMEM((K,), jnp.int32) for _ in range(2 * NIB)]
  scratch += [pltpu.SemaphoreType.DMA for _ in range(2 * NIB)]
  scratch += [pltpu.VMEM((K, D), _f32) for _ in range(NBUF)]
  scratch += [pltpu.SemaphoreType.DMA for _ in range(NBUF)]
  if with_ea:
    scratch += [pltpu.VMEM_SHARED((NP, EAP), _f32)]
    scratch += [pltpu.VMEM((K, EAP), _f32) for _ in range(NBUF)]
    scratch += [pltpu.SemaphoreType.DMA for _ in range(NBUF)]
  if with_att:
    scratch += [pltpu.VMEM((K, EAP), _f32) for _ in range(NBUF)]
    scratch += [pltpu.SemaphoreType.DMA for _ in range(NBUF)]

  def body(*refs):
    it = iter(refs)
    y = next(it)
    src_r = next(it)
    dst_r = next(it)
    zeros = next(it)
    ea = next(it) if with_ea else None
    z16 = next(it) if with_ea else None
    att = next(it) if with_att else None
    s_out = next(it)
    ea_out = next(it) if with_ea else None
    acc = next(it)
    isrc = [next(it) for _ in range(NIB)]
    idst = [next(it) for _ in range(NIB)]
    sem_is = [next(it) for _ in range(NIB)]
    sem_id = [next(it) for _ in range(NIB)]
    rows = [next(it) for _ in range(NBUF)]
    sem_r = [next(it) for _ in range(NBUF)]
    if with_ea:
      acc_ea = next(it)
      eab = [next(it) for _ in range(NBUF)]
      sem_e = [next(it) for _ in range(NBUF)]
    if with_att:
      attb = [next(it) for _ in range(NBUF)]
      sem_a = [next(it) for _ in range(NBUF)]

    c = lax.axis_index("c")
    s = lax.axis_index("s")
    wid = s * NC + c
    r0 = s * ZR

    def start_idx(m, r):
      row = wid * NCHUNK + m * GW + r
      p = r % NIB
      return [pltpu.async_copy(src_r.at[row], isrc[p], sem_is[p]),
              pltpu.async_copy(dst_r.at[row], idst[p], sem_id[p])]

    def start_g(m, r):
      b = r % NBUF
      idx = isrc[r % NIB]
      ds = [pltpu.async_copy(y.at[idx], rows[b], sem_r[b])]
      if with_att:
        ds.append(pltpu.async_copy(att.at[idx], attb[b], sem_a[b]))
      if with_ea:
        ds.append(pltpu.async_copy(
            ea.at[pl.ds(wid * EPW + m * (GW * K) + r * K, K)],
            eab[b], sem_e[b]))
      return ds

    def finish(r, ds):
      b = r % NBUF
      for d in ds:
        d.wait()
      if with_att:
        def mul8(t, carry):
          for jj in range(8):
            j = t * 8 + jj
            eab[b][j, :] = eab[b][j, :] * attb[b][j, :]
          return carry

        lax.fori_loop(0, K // 8, mul8, 0)
      didx = idst[r % NIB]
      pltpu.sync_copy(rows[b], acc.at[didx], add=True)
      if with_ea:
        pltpu.sync_copy(eab[b], acc_ea.at[didx], add=True)

    pltpu.sync_copy(zeros.at[pl.ds(r0, ZR)], acc.at[pl.ds(r0, ZR)])
    if with_ea:
      pltpu.sync_copy(z16.at[pl.ds(r0, ZR)], acc_ea.at[pl.ds(r0, ZR)])

    plsc.subcore_barrier()

    def window(m, carry):
      d_i = {0: start_idx(m, 0), 1: start_idx(m, 1)}
      ds = {}
      for r in range(GW):
        for d in d_i.pop(r):
          d.wait()
        ds[r] = start_g(m, r)
        if r == 0:
          d_i[2] = start_idx(m, 2)
        else:
          finish(r - 1, ds.pop(r - 1))
          if r + 2 < GW:
            d_i[r + 2] = start_idx(m, r + 2)
      finish(GW - 1, ds.pop(GW - 1))
      return carry

    lax.fori_loop(0, NWND, window, 0)

    plsc.subcore_barrier()
    o0 = c * NP + r0
    pltpu.sync_copy(acc.at[pl.ds(r0, ZR)], s_out.at[pl.ds(o0, ZR)])
    if with_ea:
      pltpu.sync_copy(acc_ea.at[pl.ds(r0, ZR)], ea_out.at[pl.ds(o0, ZR)])

  return pl.kernel(
      body, out_type=out_type, mesh=_sc_mesh(),
      scratch_types=scratch,
      compiler_params=pltpu.CompilerParams(use_tc_tiling_on_sc=False))


_sc_pass_plain = _make_sc_pass(with_ea=False, with_att=False)
_sc_pass_ea = _make_sc_pass(with_ea=True, with_att=False)
_sc_pass_ea_att = _make_sc_pass(with_ea=True, with_att=True)


def _dot(a, b):
  return jnp.dot(a, b, preferred_element_type=_f32)


def _part_specs():
  return [
      pl.BlockSpec((1, NB, D), lambda i: (0, i, 0)),
      pl.BlockSpec((1, NB, D), lambda i: (1, i, 0)),
      pl.BlockSpec((1, NB, EAP), lambda i: (0, i, 0)),
      pl.BlockSpec((1, NB, EAP), lambda i: (1, i, 0)),
  ]


def _w_spec(shape):
  return pl.BlockSpec(shape, lambda i: tuple(0 for _ in shape))


def _layer1_body(sa, sb, pa, pb, x, we, w, ws, b, o_h1):
  agg = sa[0] + sb[0] + _dot((pa[0] + pb[0])[:, :DE], we[...])
  o_h1[...] = jax.nn.relu(_dot(agg, w[...]) + _dot(x[...], ws[...]) + b[...])


def _layer2_body(sa, sb, pa, pb, h1, x, we, w, ws, b, we1, be1, we2, be2,
                 o_att, o_y3):
  agg = sa[0] + sb[0] + _dot((pa[0] + pb[0])[:, :DE], we[...])
  emb = jax.nn.relu(_dot(agg, w[...]) + _dot(h1[...], ws[...]) + b[...])
  logit = _dot(jax.nn.relu(_dot(emb, we1[...]) + be1[...]), we2[...]) + be2[...]
  att = jax.nn.sigmoid(logit)
  o_att[...] = jnp.broadcast_to(att, (NB, EAP))
  o_y3[...] = x[...] * att


def _layer3_body(sa, sb, pa, pb, attp, x, we, w, ws, b, o_g1, o_y4):
  att = attp[:, 0:1]
  agg = (sa[0] + sb[0] + _dot((pa[0] + pb[0])[:, :DE], we[...])) * att
  g1 = jax.nn.relu(_dot(agg, w[...]) + _dot(x[...], ws[...]) + b[...])
  o_g1[...] = g1
  o_y4[...] = g1 * att


def _layer4_body(sa, sb, pa, pb, attp, g1, we, w, ws, b, o_g2):
  att = attp[:, 0:1]
  agg = (sa[0] + sb[0] + _dot((pa[0] + pb[0])[:, :DE], we[...])) * att
  o_g2[...] = jax.nn.relu(_dot(agg, w[...]) + _dot(g1[...], ws[...]) + b[...])


def _pool_body(g2, bt, wout, bout, wm, bm, o, acc, cnt):
  i = pl.program_id(0)

  @pl.when(i == 0)
  def _():
    acc[...] = jnp.zeros_like(acc)
    cnt[...] = jnp.zeros_like(cnt)

  gids = lax.broadcasted_iota(jnp.int32, (G, NB), 0)
  mask = (bt[0, 0, :][None, :] == gids).astype(_f32)
  acc[...] += _dot(mask, g2[...])
  cnt[...] += jnp.broadcast_to(jnp.sum(mask, axis=1, keepdims=True), (G, D))

  @pl.when(i == NBLK - 1)
  def _():
    pooled = acc[...] / jnp.maximum(cnt[...], 1.0)
    logits = _dot(pooled, wout[...]) + bout[...]
    o[...] = _dot(logits, wm[...]) + bm[...]


def _tc_layer1(sp, pp, x, we, w, ws, b):
  specs = _part_specs() + [
      pl.BlockSpec((NB, D), lambda i: (i, 0)),
      _w_spec((DE, D)), _w_spec((D, D)), _w_spec((D, D)), _w_spec((1, D)),
  ]
  return pl.pallas_call(
      _layer1_body, grid=(NBLK,), in_specs=specs,
      out_specs=pl.BlockSpec((NB, D), lambda i: (i, 0)),
      out_shape=jax.ShapeDtypeStruct((N_NODES, D), _f32),
  )(sp, sp, pp, pp, x, we, w, ws, b)


def _tc_layer2(sp, pp, h1, x, we, w, ws, b, we1, be1, we2, be2):
  specs = _part_specs() + [
      pl.BlockSpec((NB, D), lambda i: (i, 0)),
      pl.BlockSpec((NB, D), lambda i: (i, 0)),
      _w_spec((DE, D)), _w_spec((D, D)), _w_spec((D, D)), _w_spec((1, D)),
      _w_spec((D, 64)), _w_spec((1, 64)), _w_spec((64, 1)), _w_spec((1, 1)),
  ]
  return pl.pallas_call(
      _layer2_body, grid=(NBLK,), in_specs=specs,
      out_specs=[pl.BlockSpec((NB, EAP), lambda i: (i, 0)),
                 pl.BlockSpec((NB, D), lambda i: (i, 0))],
      out_shape=[jax.ShapeDtypeStruct((N_NODES, EAP), _f32),
                 jax.ShapeDtypeStruct((N_NODES, D), _f32)],
  )(sp, sp, pp, pp, h1, x, we, w, ws, b, we1, be1, we2, be2)


def _tc_layer3(sp, pp, attp, x, we, w, ws, b):
  specs = _part_specs() + [
      pl.BlockSpec((NB, EAP), lambda i: (i, 0)),
      pl.BlockSpec((NB, D), lambda i: (i, 0)),
      _w_spec((DE, D)), _w_spec((D, D)), _w_spec((D, D)), _w_spec((1, D)),
  ]
  return pl.pallas_call(
      _layer3_body, grid=(NBLK,), in_specs=specs,
      out_specs=[pl.BlockSpec((NB, D), lambda i: (i, 0)),
                 pl.BlockSpec((NB, D), lambda i: (i, 0))],
      out_shape=[jax.ShapeDtypeStruct((N_NODES, D), _f32),
                 jax.ShapeDtypeStruct((N_NODES, D), _f32)],
  )(sp, sp, pp, pp, attp, x, we, w, ws, b)


def _tc_layer4(sp, pp, attp, g1, we, w, ws, b):
  specs = _part_specs() + [
      pl.BlockSpec((NB, EAP), lambda i: (i, 0)),
      pl.BlockSpec((NB, D), lambda i: (i, 0)),
      _w_spec((DE, D)), _w_spec((D, D)), _w_spec((D, D)), _w_spec((1, D)),
  ]
  return pl.pallas_call(
      _layer4_body, grid=(NBLK,), in_specs=specs,
      out_specs=pl.BlockSpec((NB, D), lambda i: (i, 0)),
      out_shape=jax.ShapeDtypeStruct((N_NODES, D), _f32),
  )(sp, sp, pp, pp, attp, g1, we, w, ws, b)


def _tc_pool(g2, batch_r, wout, bout, wm, bm):
  specs = [
      pl.BlockSpec((NB, D), lambda i: (i, 0)),
      pl.BlockSpec((1, 1, NB), lambda i: (i, 0, 0)),
      _w_spec((D, 1)), _w_spec((1, 1)), _w_spec((1, 2)), _w_spec((1, 2)),
  ]
  return pl.pallas_call(
      _pool_body, grid=(NBLK,), in_specs=specs,
      out_specs=pl.BlockSpec((G, 2), lambda i: (0, 0)),
      out_shape=jax.ShapeDtypeStruct((G, 2), _f32),
      scratch_shapes=[pltpu.VMEM((G, D), _f32), pltpu.VMEM((G, D), _f32)],
  )(g2, batch_r, wout, bout, wm, bm)


@jax.jit
def kernel(x, edge_index, batch, edge_attr, We1, W1, Ws1, b1, We2, W2, Ws2,
           b2, Wext1, bext1, Wext2, bext2, Wout, bout, Wm, bm):
  src = edge_index[0]
  dst = edge_index[1]
  epw_real = N_EDGES // NW
  dum_src = jnp.zeros((NW, NDUM), jnp.int32)
  dum_dst = jnp.broadcast_to(
      N_NODES + jnp.arange(NDUM, dtype=jnp.int32), (NW, NDUM))
  src_r = jnp.concatenate(
      [src.reshape(NW, epw_real), dum_src], axis=1).reshape(NW * NCHUNK, K)
  dst_r = jnp.concatenate(
      [dst.reshape(NW, epw_real), dum_dst], axis=1).reshape(NW * NCHUNK, K)
  ea_pad = jnp.zeros((NW, EPW, EAP), _f32).at[:, :epw_real, :DE].set(
      edge_attr.reshape(NW, epw_real, DE)).reshape(EPAD, EAP)
  zeros = jnp.zeros((NP, D), _f32)
  z16 = jnp.zeros((NP, EAP), _f32)
  b1r = b1.reshape(1, D)
  b2r = b2.reshape(1, D)
  be1 = bext1.reshape(1, 64)
  be2 = bext2.reshape(1, 1)
  boutr = bout.reshape(1, 1)
  bmr = bm.reshape(1, 2)
  batch_r = batch.reshape(NBLK, 1, NB)

  s1, p4a = _sc_pass_ea(x, src_r, dst_r, zeros, ea_pad, z16)
  s1 = s1.reshape(NC, NP, D)
  p4a = p4a.reshape(NC, NP, EAP)
  h1 = _tc_layer1(s1, p4a, x, We1, W1, Ws1, b1r)

  (s2,) = _sc_pass_plain(h1, src_r, dst_r, zeros)
  s2 = s2.reshape(NC, NP, D)
  attp, y3 = _tc_layer2(s2, p4a, h1, x, We2, W2, Ws2, b2r,
                        Wext1, be1, Wext2, be2)

  s3, p4b = _sc_pass_ea_att(y3, src_r, dst_r, zeros, ea_pad, z16, attp)
  s3 = s3.reshape(NC, NP, D)
  p4b = p4b.reshape(NC, NP, EAP)
  g1, y4 = _tc_layer3(s3, p4b, attp, x, We1, W1, Ws1, b1r)

  (s4,) = _sc_pass_plain(y4, src_r, dst_r, zeros)
  s4 = s4.reshape(NC, NP, D)
  g2 = _tc_layer4(s4, p4b, attp, g1, We2, W2, Ws2, b2r)

  return _tc_pool(g2, batch_r, Wout, boutr, Wm, bmr)


# speedup vs baseline: 1.3693x; 1.1724x over previous
import jax
import jax.numpy as jnp
from jax import lax
from jax.experimental import pallas as pl
from jax.experimental.pallas import tpu as pltpu
from jax.experimental.pallas import tpu_sc as plsc

N_NODES = 10000
N_EDGES = 320000
D = 128
DE = 4
G = 64
EAP = 16

NC = 2
NS = 16
NW = NC * NS
K = 80
NCHUNK = 125
EPW = NCHUNK * K
NP = N_NODES
ZR = 640
ZR_LAST = NP - (NS - 1) * ZR

NB = 2000
NBLK = N_NODES // NB

_f32 = jnp.float32


def _sc_mesh():
  return plsc.VectorSubcoreMesh(
      core_axis_name="c", subcore_axis_name="s", num_cores=NC,
      num_subcores=NS)


def _make_sc_pass(with_ea: bool, with_att: bool):
  out_type = [jax.ShapeDtypeStruct((NC * NP, D), _f32)]
  if with_ea:
    out_type.append(jax.ShapeDtypeStruct((NC * NP, EAP), _f32))

  scratch = [
      pltpu.VMEM((K,), jnp.int32),
      pltpu.VMEM((K,), jnp.int32),
      pltpu.VMEM((K, D), _f32),
      pltpu.VMEM_SHARED((NP, D), _f32),
      pltpu.SemaphoreType.DMA,
  ]
  if with_ea:
    scratch += [
        pltpu.VMEM((K, EAP), _f32),
        pltpu.VMEM_SHARED((NP, EAP), _f32),
    ]
  if with_att:
    scratch += [
        pltpu.VMEM((K, EAP), _f32),
        pltpu.SemaphoreType.DMA,
    ]

  def body(*refs):
    it = iter(refs)
    y = next(it)
    src = next(it)
    dst = next(it)
    zeros = next(it)
    ea = next(it) if with_ea else None
    z16 = next(it) if with_ea else None
    att = next(it) if with_att else None
    s_out = next(it)
    ea_out = next(it) if with_ea else None
    src_i = next(it)
    dst_i = next(it)
    rows = next(it)
    acc = next(it)
    sem = next(it)
    if with_ea:
      ea_b = next(it)
      acc_ea = next(it)
    if with_att:
      att_r = next(it)
      sem_a = next(it)

    c = lax.axis_index("c")
    s = lax.axis_index("s")
    wid = s * NC + c
    r0 = s * ZR

    def _zero(rows_n):
      pltpu.sync_copy(zeros.at[pl.ds(r0, rows_n)], acc.at[pl.ds(r0, rows_n)])
      if with_ea:
        pltpu.sync_copy(z16.at[pl.ds(r0, rows_n)],
                        acc_ea.at[pl.ds(r0, rows_n)])

    @pl.when(s < NS - 1)
    def _():
      _zero(ZR)

    @pl.when(s == NS - 1)
    def _():
      _zero(ZR_LAST)

    plsc.subcore_barrier()

    def chunk(i, carry):
      base = wid * EPW + i * K
      pltpu.sync_copy(src.at[pl.ds(base, K)], src_i)
      pltpu.sync_copy(dst.at[pl.ds(base, K)], dst_i)
      cp = pltpu.async_copy(y.at[src_i], rows, sem)
      if with_att:
        cp_a = pltpu.async_copy(att.at[src_i], att_r, sem_a)
      if with_ea:
        pltpu.sync_copy(ea.at[pl.ds(base, K)], ea_b)
      cp.wait()
      if with_att:
        cp_a.wait()

        def mul8(t, carry2):
          for jj in range(8):
            j = t * 8 + jj
            ea_b[j, :] = ea_b[j, :] * att_r[j, :]
          return carry2

        lax.fori_loop(0, K // 8, mul8, 0)
      pltpu.sync_copy(rows, acc.at[dst_i], add=True)
      if with_ea:
        pltpu.sync_copy(ea_b, acc_ea.at[dst_i], add=True)
      return carry

    lax.fori_loop(0, NCHUNK, chunk, 0)

    plsc.subcore_barrier()
    o0 = c * NP + r0

    def _wb(rows_n):
      pltpu.sync_copy(acc.at[pl.ds(r0, rows_n)], s_out.at[pl.ds(o0, rows_n)])
      if with_ea:
        pltpu.sync_copy(acc_ea.at[pl.ds(r0, rows_n)],
                        ea_out.at[pl.ds(o0, rows_n)])

    @pl.when(s < NS - 1)
    def _():
      _wb(ZR)

    @pl.when(s == NS - 1)
    def _():
      _wb(ZR_LAST)

  return pl.kernel(
      body, out_type=out_type, mesh=_sc_mesh(),
      scratch_types=scratch,
      compiler_params=pltpu.CompilerParams(use_tc_tiling_on_sc=False))


_sc_pass_plain = _make_sc_pass(with_ea=False, with_att=False)
_sc_pass_ea = _make_sc_pass(with_ea=True, with_att=False)
_sc_pass_ea_att = _make_sc_pass(with_ea=True, with_att=True)


def _dot(a, b):
  return jnp.dot(a, b, preferred_element_type=_f32)


def _part_specs():
  return [
      pl.BlockSpec((1, NB, D), lambda i: (0, i, 0)),
      pl.BlockSpec((1, NB, D), lambda i: (1, i, 0)),
      pl.BlockSpec((1, NB, EAP), lambda i: (0, i, 0)),
      pl.BlockSpec((1, NB, EAP), lambda i: (1, i, 0)),
  ]


def _w_spec(shape):
  return pl.BlockSpec(shape, lambda i: tuple(0 for _ in shape))


def _layer1_body(sa, sb, pa, pb, x, we, w, ws, b, o_h1):
  agg = sa[0] + sb[0] + _dot((pa[0] + pb[0])[:, :DE], we[...])
  o_h1[...] = jax.nn.relu(_dot(agg, w[...]) + _dot(x[...], ws[...]) + b[...])


def _layer2_body(sa, sb, pa, pb, h1, x, we, w, ws, b, we1, be1, we2, be2,
                 o_att, o_y3):
  agg = sa[0] + sb[0] + _dot((pa[0] + pb[0])[:, :DE], we[...])
  emb = jax.nn.relu(_dot(agg, w[...]) + _dot(h1[...], ws[...]) + b[...])
  logit = _dot(jax.nn.relu(_dot(emb, we1[...]) + be1[...]), we2[...]) + be2[...]
  att = jax.nn.sigmoid(logit)
  o_att[...] = jnp.broadcast_to(att, (NB, EAP))
  o_y3[...] = x[...] * att


def _layer3_body(sa, sb, pa, pb, attp, x, we, w, ws, b, o_g1, o_y4):
  att = attp[:, 0:1]
  agg = (sa[0] + sb[0] + _dot((pa[0] + pb[0])[:, :DE], we[...])) * att
  g1 = jax.nn.relu(_dot(agg, w[...]) + _dot(x[...], ws[...]) + b[...])
  o_g1[...] = g1
  o_y4[...] = g1 * att


def _layer4_body(sa, sb, pa, pb, attp, g1, we, w, ws, b, o_g2):
  att = attp[:, 0:1]
  agg = (sa[0] + sb[0] + _dot((pa[0] + pb[0])[:, :DE], we[...])) * att
  o_g2[...] = jax.nn.relu(_dot(agg, w[...]) + _dot(g1[...], ws[...]) + b[...])


def _pool_body(g2, bt, wout, bout, wm, bm, o, acc, cnt):
  i = pl.program_id(0)

  @pl.when(i == 0)
  def _():
    acc[...] = jnp.zeros_like(acc)
    cnt[...] = jnp.zeros_like(cnt)

  gids = lax.broadcasted_iota(jnp.int32, (G, NB), 0)
  mask = (bt[0, 0, :][None, :] == gids).astype(_f32)
  acc[...] += _dot(mask, g2[...])
  cnt[...] += jnp.broadcast_to(jnp.sum(mask, axis=1, keepdims=True), (G, D))

  @pl.when(i == NBLK - 1)
  def _():
    pooled = acc[...] / jnp.maximum(cnt[...], 1.0)
    logits = _dot(pooled, wout[...]) + bout[...]
    o[...] = _dot(logits, wm[...]) + bm[...]


def _tc_layer1(sp, pp, x, we, w, ws, b):
  specs = _part_specs() + [
      pl.BlockSpec((NB, D), lambda i: (i, 0)),
      _w_spec((DE, D)), _w_spec((D, D)), _w_spec((D, D)), _w_spec((1, D)),
  ]
  return pl.pallas_call(
      _layer1_body, grid=(NBLK,), in_specs=specs,
      out_specs=pl.BlockSpec((NB, D), lambda i: (i, 0)),
      out_shape=jax.ShapeDtypeStruct((N_NODES, D), _f32),
  )(sp, sp, pp, pp, x, we, w, ws, b)


def _tc_layer2(sp, pp, h1, x, we, w, ws, b, we1, be1, we2, be2):
  specs = _part_specs() + [
      pl.BlockSpec((NB, D), lambda i: (i, 0)),
      pl.BlockSpec((NB, D), lambda i: (i, 0)),
      _w_spec((DE, D)), _w_spec((D, D)), _w_spec((D, D)), _w_spec((1, D)),
      _w_spec((D, 64)), _w_spec((1, 64)), _w_spec((64, 1)), _w_spec((1, 1)),
  ]
  return pl.pallas_call(
      _layer2_body, grid=(NBLK,), in_specs=specs,
      out_specs=[pl.BlockSpec((NB, EAP), lambda i: (i, 0)),
                 pl.BlockSpec((NB, D), lambda i: (i, 0))],
      out_shape=[jax.ShapeDtypeStruct((N_NODES, EAP), _f32),
                 jax.ShapeDtypeStruct((N_NODES, D), _f32)],
  )(sp, sp, pp, pp, h1, x, we, w, ws, b, we1, be1, we2, be2)


def _tc_layer3(sp, pp, attp, x, we, w, ws, b):
  specs = _part_specs() + [
      pl.BlockSpec((NB, EAP), lambda i: (i, 0)),
      pl.BlockSpec((NB, D), lambda i: (i, 0)),
      _w_spec((DE, D)), _w_spec((D, D)), _w_spec((D, D)), _w_spec((1, D)),
  ]
  return pl.pallas_call(
      _layer3_body, grid=(NBLK,), in_specs=specs,
      out_specs=[pl.BlockSpec((NB, D), lambda i: (i, 0)),
                 pl.BlockSpec((NB, D), lambda i: (i, 0))],
      out_shape=[jax.ShapeDtypeStruct((N_NODES, D), _f32),
                 jax.ShapeDtypeStruct((N_NODES, D), _f32)],
  )(sp, sp, pp, pp, attp, x, we, w, ws, b)


def _tc_layer4(sp, pp, attp, g1, we, w, ws, b):
  specs = _part_specs() + [
      pl.BlockSpec((NB, EAP), lambda i: (i, 0)),
      pl.BlockSpec((NB, D), lambda i: (i, 0)),
      _w_spec((DE, D)), _w_spec((D, D)), _w_spec((D, D)), _w_spec((1, D)),
  ]
  return pl.pallas_call(
      _layer4_body, grid=(NBLK,), in_specs=specs,
      out_specs=pl.BlockSpec((NB, D), lambda i: (i, 0)),
      out_shape=jax.ShapeDtypeStruct((N_NODES, D), _f32),
  )(sp, sp, pp, pp, attp, g1, we, w, ws, b)


def _tc_pool(g2, batch_r, wout, bout, wm, bm):
  specs = [
      pl.BlockSpec((NB, D), lambda i: (i, 0)),
      pl.BlockSpec((1, 1, NB), lambda i: (i, 0, 0)),
      _w_spec((D, 1)), _w_spec((1, 1)), _w_spec((1, 2)), _w_spec((1, 2)),
  ]
  return pl.pallas_call(
      _pool_body, grid=(NBLK,), in_specs=specs,
      out_specs=pl.BlockSpec((G, 2), lambda i: (0, 0)),
      out_shape=jax.ShapeDtypeStruct((G, 2), _f32),
      scratch_shapes=[pltpu.VMEM((G, D), _f32), pltpu.VMEM((G, D), _f32)],
  )(g2, batch_r, wout, bout, wm, bm)


@jax.jit
def kernel(x, edge_index, batch, edge_attr, We1, W1, Ws1, b1, We2, W2, Ws2,
           b2, Wext1, bext1, Wext2, bext2, Wout, bout, Wm, bm):
  src_r = edge_index[0]
  dst_r = edge_index[1]
  ea_pad = jnp.zeros((N_EDGES, EAP), _f32).at[:, :DE].set(edge_attr)
  zeros = jnp.zeros((NP, D), _f32)
  z16 = jnp.zeros((NP, EAP), _f32)
  b1r = b1.reshape(1, D)
  b2r = b2.reshape(1, D)
  be1 = bext1.reshape(1, 64)
  be2 = bext2.reshape(1, 1)
  boutr = bout.reshape(1, 1)
  bmr = bm.reshape(1, 2)
  batch_r = batch.reshape(NBLK, 1, NB)

  s1, p4a = _sc_pass_ea(x, src_r, dst_r, zeros, ea_pad, z16)
  s1 = s1.reshape(NC, NP, D)
  p4a = p4a.reshape(NC, NP, EAP)
  h1 = _tc_layer1(s1, p4a, x, We1, W1, Ws1, b1r)

  (s2,) = _sc_pass_plain(h1, src_r, dst_r, zeros)
  s2 = s2.reshape(NC, NP, D)
  attp, y3 = _tc_layer2(s2, p4a, h1, x, We2, W2, Ws2, b2r,
                        Wext1, be1, Wext2, be2)

  s3, p4b = _sc_pass_ea_att(y3, src_r, dst_r, zeros, ea_pad, z16, attp)
  s3 = s3.reshape(NC, NP, D)
  p4b = p4b.reshape(NC, NP, EAP)
  g1, y4 = _tc_layer3(s3, p4b, attp, x, We1, W1, Ws1, b1r)

  (s4,) = _sc_pass_plain(y4, src_r, dst_r, zeros)
  s4 = s4.reshape(NC, NP, D)
  g2 = _tc_layer4(s4, p4b, attp, g1, We2, W2, Ws2, b2r)

  return _tc_pool(g2, batch_r, Wout, boutr, Wm, bmr)
